# Initial kernel scaffold; baseline (speedup 1.0000x reference)
import jax
import jax.numpy as jnp
from jax.experimental import pallas as pl


def kernel(coeffs, ground_state_coeff_mean, basis_function_ind, coeffs_batch):
    raise NotImplementedError("write your pallas kernel here")


# speedup vs baseline: 173.6602x; 173.6602x over previous
import functools

import jax
import jax.numpy as jnp
from jax import lax
from jax.experimental import pallas as pl
from jax.experimental.pallas import tpu as pltpu
from jax.experimental.pallas import tpu_sc as plsc

N_ELEMS = 6_400_000
N_TYPES = 128
N_SEG = 512
FACTOR = 0.5

NC = 2
NS = 16
L = 16
NW = NC * NS
PER_W = N_ELEMS // NW
BLK = 20_000
NBLK = PER_W // BLK


def _sc_partials(coeffs, mean, ind, batch):
    mesh = plsc.VectorSubcoreMesh(core_axis_name="c", subcore_axis_name="s")

    @functools.partial(
        pl.kernel,
        mesh=mesh,
        out_type=jax.ShapeDtypeStruct((NW, N_SEG), jnp.float32),
        compiler_params=pltpu.CompilerParams(needs_layout_passes=False),
        scratch_types=[
            pltpu.VMEM((N_TYPES,), jnp.float32),
            pltpu.VMEM((BLK,), jnp.float32),
            pltpu.VMEM((BLK,), jnp.int32),
            pltpu.VMEM((BLK,), jnp.int32),
            pltpu.VMEM((N_SEG,), jnp.float32),
        ],
    )
    def k(coeffs_hbm, mean_hbm, ind_hbm, batch_hbm, out_hbm,
          mean_v, c_v, i_v, b_v, acc_v):
        wid = lax.axis_index("s") * NC + lax.axis_index("c")
        base = pl.multiple_of(wid * PER_W, 8)

        pltpu.sync_copy(mean_hbm, mean_v)
        zeros = jnp.zeros((L,), jnp.float32)
        for j in range(N_SEG // L):
            acc_v[pl.ds(j * L, L)] = zeros

        def blk_body(kk, carry):
            off = pl.multiple_of(base + kk * BLK, 8)
            pltpu.sync_copy(coeffs_hbm.at[pl.ds(off, BLK)], c_v)
            pltpu.sync_copy(ind_hbm.at[pl.ds(off, BLK)], i_v)
            pltpu.sync_copy(batch_hbm.at[pl.ds(off, BLK)], b_v)

            def body(i, c2):
                o = i * L
                c = c_v[pl.ds(o, L)]
                m = plsc.load_gather(mean_v, [i_v[pl.ds(o, L)]])
                d = c - m
                plsc.addupdate_scatter(acc_v, [b_v[pl.ds(o, L)]], d * d)
                return c2

            lax.fori_loop(0, BLK // L, body, 0, unroll=8)
            return carry

        lax.fori_loop(0, NBLK, blk_body, 0)
        pltpu.sync_copy(acc_v, out_hbm.at[wid])

    return k(coeffs, mean, ind, batch)


def _tc_combine(partials):
    def body(p_ref, o_ref):
        o_ref[...] = FACTOR * jnp.sum(p_ref[...], axis=0, keepdims=True)

    out = pl.pallas_call(
        body,
        out_shape=jax.ShapeDtypeStruct((1, N_SEG), jnp.float32),
    )(partials)
    return out[0]


def kernel(coeffs, ground_state_coeff_mean, basis_function_ind, coeffs_batch):
    ind = basis_function_ind.astype(jnp.int32)
    batch = coeffs_batch.astype(jnp.int32)
    partials = _sc_partials(coeffs, ground_state_coeff_mean, ind, batch)
    return _tc_combine(partials)


# speedup vs baseline: 256.2125x; 1.4754x over previous
import functools

import jax
import jax.numpy as jnp
from jax import lax
from jax.experimental import pallas as pl
from jax.experimental.pallas import tpu as pltpu
from jax.experimental.pallas import tpu_sc as plsc

N_ELEMS = 6_400_000
N_TYPES = 128
N_SEG = 512
FACTOR = 0.5

NC = 2
NS = 16
L = 16
NW = NC * NS
PER_W = N_ELEMS // NW
BLK = 20_000
NBLK = PER_W // BLK


def _sc_partials(coeffs, mean, ind, batch):
    mesh = plsc.VectorSubcoreMesh(core_axis_name="c", subcore_axis_name="s")

    @functools.partial(
        pl.kernel,
        mesh=mesh,
        out_type=jax.ShapeDtypeStruct((NW, N_SEG), jnp.float32),
        compiler_params=pltpu.CompilerParams(needs_layout_passes=False),
        scratch_types=[
            pltpu.VMEM((N_TYPES,), jnp.float32),
            pltpu.VMEM((BLK,), jnp.float32),
            pltpu.VMEM((BLK,), jnp.int32),
            pltpu.VMEM((BLK,), jnp.int32),
            pltpu.VMEM((BLK,), jnp.float32),
            pltpu.VMEM((BLK,), jnp.int32),
            pltpu.VMEM((BLK,), jnp.int32),
            pltpu.VMEM((N_SEG,), jnp.float32),
            pltpu.SemaphoreType.DMA,
            pltpu.SemaphoreType.DMA,
        ],
    )
    def k(coeffs_hbm, mean_hbm, ind_hbm, batch_hbm, out_hbm,
          mean_v, c0, i0, b0, c1, i1, b1, acc_v, sem0, sem1):
        wid = lax.axis_index("s") * NC + lax.axis_index("c")
        base = pl.multiple_of(wid * PER_W, 8)
        bufs = ((c0, i0, b0, sem0), (c1, i1, b1, sem1))

        def start(kk):
            c_v, i_v, b_v, sem = bufs[kk % 2]
            off = pl.multiple_of(base + kk * BLK, 8)
            return (
                pltpu.async_copy(coeffs_hbm.at[pl.ds(off, BLK)], c_v, sem),
                pltpu.async_copy(ind_hbm.at[pl.ds(off, BLK)], i_v, sem),
                pltpu.async_copy(batch_hbm.at[pl.ds(off, BLK)], b_v, sem),
            )

        handles = start(0)
        pltpu.sync_copy(mean_hbm, mean_v)
        zeros = jnp.zeros((L,), jnp.float32)
        for j in range(N_SEG // L):
            acc_v[pl.ds(j * L, L)] = zeros

        for kk in range(NBLK):
            for h in handles:
                h.wait()
            c_v, i_v, b_v, _ = bufs[kk % 2]
            if kk + 1 < NBLK:
                handles = start(kk + 1)

            @plsc.parallel_loop(0, BLK // L, unroll=8)
            def body(ii):
                o = ii * L
                c = c_v[pl.ds(o, L)]
                m = plsc.load_gather(mean_v, [i_v[pl.ds(o, L)]])
                d = c - m
                plsc.addupdate_scatter(acc_v, [b_v[pl.ds(o, L)]], d * d)

        pltpu.sync_copy(acc_v, out_hbm.at[wid])

    return k(coeffs, mean, ind, batch)


def _tc_combine(partials):
    def body(p_ref, o_ref):
        o_ref[...] = FACTOR * jnp.sum(p_ref[...], axis=0, keepdims=True)

    out = pl.pallas_call(
        body,
        out_shape=jax.ShapeDtypeStruct((1, N_SEG), jnp.float32),
    )(partials)
    return out[0]


def kernel(coeffs, ground_state_coeff_mean, basis_function_ind, coeffs_batch):
    ind = basis_function_ind.astype(jnp.int32)
    batch = coeffs_batch.astype(jnp.int32)
    partials = _sc_partials(coeffs, ground_state_coeff_mean, ind, batch)
    return _tc_combine(partials)


# speedup vs baseline: 1024.1578x; 3.9973x over previous
import functools

import jax
import jax.numpy as jnp
from jax import lax
from jax.experimental import pallas as pl
from jax.experimental.pallas import tpu as pltpu
from jax.experimental.pallas import tpu_sc as plsc

N_ELEMS = 6_400_000
N_TYPES = 128
N_SEG = 512
FACTOR = 0.5

NC = 2
NS = 16
L = 16
NW = NC * NS
PER_W = N_ELEMS // NW
BLK = 20_000
NBLK = PER_W // BLK


def _sc_partials(coeffs, mean, ind, batch):
    mesh = plsc.VectorSubcoreMesh(core_axis_name="c", subcore_axis_name="s")

    @functools.partial(
        pl.kernel,
        mesh=mesh,
        out_type=jax.ShapeDtypeStruct((NW, N_SEG), jnp.float32),
        compiler_params=pltpu.CompilerParams(needs_layout_passes=False),
        scratch_types=[
            pltpu.VMEM((N_TYPES,), jnp.float32),
            pltpu.VMEM((BLK,), jnp.float32),
            pltpu.VMEM((BLK,), jnp.int32),
            pltpu.VMEM((BLK,), jnp.int32),
            pltpu.VMEM((BLK,), jnp.float32),
            pltpu.VMEM((BLK,), jnp.int32),
            pltpu.VMEM((BLK,), jnp.int32),
            pltpu.VMEM((N_SEG * L,), jnp.float32),
            pltpu.VMEM((N_SEG,), jnp.float32),
            pltpu.SemaphoreType.DMA,
            pltpu.SemaphoreType.DMA,
        ],
    )
    def k(coeffs_hbm, mean_hbm, ind_hbm, batch_hbm, out_hbm,
          mean_v, c0, i0, b0, c1, i1, b1, acc2_v, acc_v, sem0, sem1):
        wid = lax.axis_index("s") * NC + lax.axis_index("c")
        base = pl.multiple_of(wid * PER_W, 8)
        bufs = ((c0, i0, b0, sem0), (c1, i1, b1, sem1))

        def start(kk):
            c_v, i_v, b_v, sem = bufs[kk % 2]
            off = pl.multiple_of(base + kk * BLK, 8)
            return (
                pltpu.async_copy(coeffs_hbm.at[pl.ds(off, BLK)], c_v, sem),
                pltpu.async_copy(ind_hbm.at[pl.ds(off, BLK)], i_v, sem),
                pltpu.async_copy(batch_hbm.at[pl.ds(off, BLK)], b_v, sem),
            )

        handles = start(0)
        pltpu.sync_copy(mean_hbm, mean_v)
        zeros = jnp.zeros((L,), jnp.float32)

        @plsc.parallel_loop(0, N_SEG, unroll=8)
        def zero_body(s):
            acc2_v[pl.ds(s * L, L)] = zeros

        lane = lax.iota(jnp.int32, L)

        for kk in range(NBLK):
            for h in handles:
                h.wait()
            c_v, i_v, b_v, _ = bufs[kk % 2]
            if kk + 1 < NBLK:
                handles = start(kk + 1)

            @plsc.parallel_loop(0, BLK // L, unroll=8)
            def body(ii):
                o = ii * L
                c = c_v[pl.ds(o, L)]
                m = plsc.load_gather(mean_v, [i_v[pl.ds(o, L)]])
                d = c - m
                idx = (b_v[pl.ds(o, L)] << 4) + lane
                plsc.addupdate_scatter(acc2_v, [idx], d * d)

        lane0 = lane == 0

        @plsc.parallel_loop(0, N_SEG, unroll=8)
        def fold_body(s):
            row = acc2_v[pl.ds(s * L, L)]
            tot = jnp.broadcast_to(jnp.sum(row), (L,))
            plsc.store_scatter(acc_v, [jnp.broadcast_to(s, (L,))], tot,
                               mask=lane0)

        pltpu.sync_copy(acc_v, out_hbm.at[wid])

    return k(coeffs, mean, ind, batch)


def _tc_combine(partials):
    def body(p_ref, o_ref):
        o_ref[...] = FACTOR * jnp.sum(p_ref[...], axis=0, keepdims=True)

    out = pl.pallas_call(
        body,
        out_shape=jax.ShapeDtypeStruct((1, N_SEG), jnp.float32),
    )(partials)
    return out[0]


def kernel(coeffs, ground_state_coeff_mean, basis_function_ind, coeffs_batch):
    ind = basis_function_ind.astype(jnp.int32)
    batch = coeffs_batch.astype(jnp.int32)
    partials = _sc_partials(coeffs, ground_state_coeff_mean, ind, batch)
    return _tc_combine(partials)
